# baseline (speedup 1.0000x reference)
import functools

import jax
import jax.numpy as jnp
from jax import lax
from jax.experimental import pallas as pl
from jax.experimental.pallas import tpu as pltpu
from jax.experimental.pallas import tpu_sc as plsc

BATCH = 16384
FIELDS = 26
EMBED_DIM = 32
VOCAB = 1000000
B = BATCH * FIELDS

NUM_CORES = 2
NUM_SUBCORES = 16
NW = NUM_CORES * NUM_SUBCORES
BPW = B // NW
CHUNK = 512
NCHUNK = BPW // CHUNK
GBUF = 4

BLK = 128
NBB = BATCH // BLK
BB_PER_W = NBB // NW
TILES_PER_W = FIELDS * BB_PER_W
NBUF = 4

_mesh = plsc.VectorSubcoreMesh(
    core_axis_name="c",
    subcore_axis_name="s",
    num_cores=NUM_CORES,
    num_subcores=NUM_SUBCORES,
)


@functools.partial(
    pl.kernel,
    mesh=_mesh,
    out_type=jax.ShapeDtypeStruct((B, EMBED_DIM), jnp.float32),
    scratch_types=[
        pltpu.VMEM((BPW,), jnp.int32),
        [pltpu.VMEM((CHUNK, EMBED_DIM), jnp.float32) for _ in range(GBUF)],
        [pltpu.SemaphoreType.DMA for _ in range(GBUF)],
        [pltpu.SemaphoreType.DMA for _ in range(GBUF)],
    ],
    compiler_params=pltpu.CompilerParams(use_tc_tiling_on_sc=False),
)
def _gather_kernel(table_hbm, idx_hbm, out_hbm, idx_v, bufs, gsems, ssems):
    wid = lax.axis_index("s") * NUM_CORES + lax.axis_index("c")
    base = wid * BPW
    pltpu.sync_copy(idx_hbm.at[pl.ds(base, BPW)], idx_v)

    def start_gather(j, b):
        return pltpu.async_copy(
            table_hbm.at[idx_v.at[pl.ds(j * CHUNK, CHUNK)]], bufs[b], gsems[b]
        )

    gathers = {}
    stores = {}
    for b in range(GBUF):
        gathers[b] = start_gather(b, b)
    for j in range(NCHUNK):
        b = j % GBUF
        gathers[b].wait()
        stores[b] = pltpu.async_copy(
            bufs[b], out_hbm.at[pl.ds(base + j * CHUNK, CHUNK)], ssems[b]
        )
        g = j + GBUF
        if g < NCHUNK:
            stores[b].wait()
            gathers[b] = start_gather(g, b)
    for j in range(NCHUNK - GBUF, NCHUNK):
        stores[j % GBUF].wait()


@functools.partial(
    pl.kernel,
    mesh=_mesh,
    out_type=jax.ShapeDtypeStruct((FIELDS, 4, NBB, 8, BLK), jnp.float32),
    scratch_types=[
        [pltpu.VMEM((BLK, EMBED_DIM), jnp.float32) for _ in range(NBUF)],
        pltpu.VMEM((4, 8, BLK), jnp.float32),
        [pltpu.SemaphoreType.DMA for _ in range(NBUF)],
    ],
    compiler_params=pltpu.CompilerParams(
        use_tc_tiling_on_sc=False, needs_layout_passes=False
    ),
)
def _transpose_kernel(mid_hbm, out_hbm, gbufs, trans, gsems):
    wid = lax.axis_index("s") * NUM_CORES + lax.axis_index("c")
    bidx = [lax.iota(jnp.int32, 16) + g * 16 for g in range(8)]

    def body(t0, carry):
        loads = []
        for b in range(NBUF):
            t = t0 + b
            f = t // BB_PER_W
            bb = wid * BB_PER_W + t % BB_PER_W
            loads.append(
                pltpu.async_copy(mid_hbm.at[f, bb], gbufs[b], gsems[b])
            )
        for b in range(NBUF):
            t = t0 + b
            f = t // BB_PER_W
            bb = wid * BB_PER_W + t % BB_PER_W
            loads[b].wait()
            gbuf = gbufs[b]

            @plsc.parallel_loop(0, EMBED_DIM, step=1, unroll=8)
            def _transpose(e):
                ev = jnp.full((16,), 1, jnp.int32) * e
                eb_i = lax.shift_right_logical(e, 3)
                es_i = jnp.bitwise_and(e, 7)
                for g in range(8):
                    trans[eb_i, es_i, pl.ds(g * 16, 16)] = plsc.load_gather(
                        gbuf, [bidx[g], ev]
                    )

            pltpu.sync_copy(trans, out_hbm.at[f, :, bb])
        return carry

    lax.fori_loop(0, TILES_PER_W // NBUF, lambda i, c: body(i * NBUF, c), 0)


def kernel(ids, table):
    idsf = ids.T.reshape(-1)
    mid = _gather_kernel(table, idsf)
    mid4 = mid.reshape(FIELDS, NBB, BLK, EMBED_DIM)
    out5 = _transpose_kernel(mid4)
    return out5.transpose(2, 4, 0, 1, 3).reshape(BATCH, FIELDS, EMBED_DIM)


# speedup vs baseline: 1.0016x; 1.0016x over previous
import functools

import jax
import jax.numpy as jnp
from jax import lax
from jax.experimental import pallas as pl
from jax.experimental.pallas import tpu as pltpu
from jax.experimental.pallas import tpu_sc as plsc

BATCH = 16384
FIELDS = 26
EMBED_DIM = 32
VOCAB = 1000000
B = BATCH * FIELDS

NUM_CORES = 2
NUM_SUBCORES = 16
NW = NUM_CORES * NUM_SUBCORES
BPW = B // NW
CHUNK = 512
NCHUNK = BPW // CHUNK
GBUF = 4

BLK = 128
NBB = BATCH // BLK
BB_PER_W = NBB // NW
TILES_PER_W = FIELDS * BB_PER_W
NBUF = 4

_mesh = plsc.VectorSubcoreMesh(
    core_axis_name="c",
    subcore_axis_name="s",
    num_cores=NUM_CORES,
    num_subcores=NUM_SUBCORES,
)


@functools.partial(
    pl.kernel,
    mesh=_mesh,
    out_type=jax.ShapeDtypeStruct((B,), jnp.int32),
    scratch_types=[
        [pltpu.VMEM((BB_PER_W * BLK,), jnp.int32) for _ in range(2)],
        [pltpu.SemaphoreType.DMA for _ in range(2)],
        [pltpu.SemaphoreType.DMA for _ in range(2)],
    ],
)
def _ids_detile_kernel(idsT_hbm, out_hbm, bufs, lsems, ssems):
    wid = lax.axis_index("s") * NUM_CORES + lax.axis_index("c")
    col = wid * BB_PER_W * BLK
    loads = {}
    stores = {}
    for f in range(2):
        loads[f] = pltpu.async_copy(
            idsT_hbm.at[f, pl.ds(col, BB_PER_W * BLK)], bufs[f], lsems[f]
        )
    for f in range(FIELDS):
        b = f % 2
        loads[f].wait()
        stores[b] = pltpu.async_copy(
            bufs[b],
            out_hbm.at[pl.ds(f * BATCH + col, BB_PER_W * BLK)],
            ssems[b],
        )
        nf = f + 2
        if nf < FIELDS:
            stores[b].wait()
            loads[nf] = pltpu.async_copy(
                idsT_hbm.at[nf, pl.ds(col, BB_PER_W * BLK)], bufs[b], lsems[b]
            )
    for f in range(FIELDS - 2, FIELDS):
        stores[f % 2].wait()


@functools.partial(
    pl.kernel,
    mesh=_mesh,
    out_type=jax.ShapeDtypeStruct((B, EMBED_DIM), jnp.float32),
    scratch_types=[
        pltpu.VMEM((BPW,), jnp.int32),
        [pltpu.VMEM((CHUNK, EMBED_DIM), jnp.float32) for _ in range(GBUF)],
        [pltpu.SemaphoreType.DMA for _ in range(GBUF)],
        [pltpu.SemaphoreType.DMA for _ in range(GBUF)],
    ],
    compiler_params=pltpu.CompilerParams(use_tc_tiling_on_sc=False),
)
def _gather_kernel(table_hbm, idx_hbm, out_hbm, idx_v, bufs, gsems, ssems):
    wid = lax.axis_index("s") * NUM_CORES + lax.axis_index("c")
    base = wid * BPW
    pltpu.sync_copy(idx_hbm.at[pl.ds(base, BPW)], idx_v)

    def start_gather(j, b):
        return pltpu.async_copy(
            table_hbm.at[idx_v.at[pl.ds(j * CHUNK, CHUNK)]], bufs[b], gsems[b]
        )

    gathers = {}
    stores = {}
    for b in range(GBUF):
        gathers[b] = start_gather(b, b)
    for j in range(NCHUNK):
        b = j % GBUF
        gathers[b].wait()
        stores[b] = pltpu.async_copy(
            bufs[b], out_hbm.at[pl.ds(base + j * CHUNK, CHUNK)], ssems[b]
        )
        g = j + GBUF
        if g < NCHUNK:
            stores[b].wait()
            gathers[b] = start_gather(g, b)
    for j in range(NCHUNK - GBUF, NCHUNK):
        stores[j % GBUF].wait()


@functools.partial(
    pl.kernel,
    mesh=_mesh,
    out_type=jax.ShapeDtypeStruct((FIELDS, 4, NBB, 8, BLK), jnp.float32),
    scratch_types=[
        [pltpu.VMEM((BLK, EMBED_DIM), jnp.float32) for _ in range(NBUF)],
        pltpu.VMEM((4, 8, BLK), jnp.float32),
        [pltpu.SemaphoreType.DMA for _ in range(NBUF)],
    ],
    compiler_params=pltpu.CompilerParams(
        use_tc_tiling_on_sc=False, needs_layout_passes=False
    ),
)
def _transpose_kernel(mid_hbm, out_hbm, gbufs, trans, gsems):
    wid = lax.axis_index("s") * NUM_CORES + lax.axis_index("c")
    bidx = [lax.iota(jnp.int32, 16) + g * 16 for g in range(8)]

    def body(t0, carry):
        loads = []
        for b in range(NBUF):
            t = t0 + b
            f = t // BB_PER_W
            bb = wid * BB_PER_W + t % BB_PER_W
            loads.append(
                pltpu.async_copy(mid_hbm.at[f, bb], gbufs[b], gsems[b])
            )
        for b in range(NBUF):
            t = t0 + b
            f = t // BB_PER_W
            bb = wid * BB_PER_W + t % BB_PER_W
            loads[b].wait()
            gbuf = gbufs[b]

            @plsc.parallel_loop(0, EMBED_DIM, step=1, unroll=8)
            def _transpose(e):
                ev = jnp.full((16,), 1, jnp.int32) * e
                eb_i = lax.shift_right_logical(e, 3)
                es_i = jnp.bitwise_and(e, 7)
                for g in range(8):
                    trans[eb_i, es_i, pl.ds(g * 16, 16)] = plsc.load_gather(
                        gbuf, [bidx[g], ev]
                    )

            pltpu.sync_copy(trans, out_hbm.at[f, :, bb])
        return carry

    lax.fori_loop(0, TILES_PER_W // NBUF, lambda i, c: body(i * NBUF, c), 0)


def kernel(ids, table):
    idsf = _ids_detile_kernel(ids.T)
    mid = _gather_kernel(table, idsf)
    mid4 = mid.reshape(FIELDS, NBB, BLK, EMBED_DIM)
    out5 = _transpose_kernel(mid4)
    return out5.transpose(2, 4, 0, 1, 3).reshape(BATCH, FIELDS, EMBED_DIM)


# speedup vs baseline: 1.0664x; 1.0648x over previous
import functools

import jax
import jax.numpy as jnp
from jax import lax
from jax.experimental import pallas as pl
from jax.experimental.pallas import tpu as pltpu
from jax.experimental.pallas import tpu_sc as plsc

BATCH = 16384
FIELDS = 26
EMBED_DIM = 32
VOCAB = 1000000

NUM_CORES = 2
NUM_SUBCORES = 16
NW = NUM_CORES * NUM_SUBCORES
BLK = 128
NBB = BATCH // BLK
BB_PER_W = NBB // NW
IDS_PER_W = BB_PER_W * BLK
TILES_PER_W = FIELDS * BB_PER_W
NBUF = 4

_mesh = plsc.VectorSubcoreMesh(
    core_axis_name="c",
    subcore_axis_name="s",
    num_cores=NUM_CORES,
    num_subcores=NUM_SUBCORES,
)


@functools.partial(
    pl.kernel,
    mesh=_mesh,
    out_type=jax.ShapeDtypeStruct((FIELDS, 4, NBB, 8, BLK), jnp.float32),
    scratch_types=[
        pltpu.VMEM((FIELDS * IDS_PER_W,), jnp.int32),
        [pltpu.VMEM((BLK, EMBED_DIM), jnp.float32) for _ in range(NBUF)],
        [pltpu.VMEM((4, 8, BLK), jnp.float32) for _ in range(2)],
        [pltpu.SemaphoreType.DMA for _ in range(NBUF)],
        [pltpu.SemaphoreType.DMA for _ in range(2)],
    ],
    compiler_params=pltpu.CompilerParams(
        use_tc_tiling_on_sc=False, needs_layout_passes=False
    ),
)
def _gather_kernel(t_hbm, idsf_hbm, out_hbm, idsv, gbufs, transb, gsems, ssems):
    wid = lax.axis_index("s") * NUM_CORES + lax.axis_index("c")
    for f in range(FIELDS):
        pltpu.sync_copy(
            idsf_hbm.at[pl.ds(f * BATCH + wid * IDS_PER_W, IDS_PER_W)],
            idsv.at[pl.ds(f * IDS_PER_W, IDS_PER_W)],
        )

    bidx = [lax.iota(jnp.int32, 16) + g * 16 for g in range(8)]
    eidx = [jnp.full((16,), e, jnp.int32) for e in range(EMBED_DIM)]

    def body(t0, carry):
        gathers = []
        for b in range(NBUF):
            t = t0 + b
            f = t // BB_PER_W
            bl = t % BB_PER_W
            gathers.append(
                pltpu.async_copy(
                    t_hbm.at[idsv.at[pl.ds(f * IDS_PER_W + bl * BLK, BLK)]],
                    gbufs[b],
                    gsems[b],
                )
            )
        stores = {}
        for b in range(NBUF):
            t = t0 + b
            f = t // BB_PER_W
            bb = wid * BB_PER_W + t % BB_PER_W
            gathers[b].wait()
            gbuf = gbufs[b]
            trans = transb[b % 2]
            if b >= 2:
                stores[b - 2].wait()

            @plsc.parallel_loop(0, EMBED_DIM, step=1, unroll=8)
            def _transpose(e):
                ev = jnp.full((16,), 1, jnp.int32) * e
                eb_i = lax.shift_right_logical(e, 3)
                es_i = jnp.bitwise_and(e, 7)
                for g in range(8):
                    trans[eb_i, es_i, pl.ds(g * 16, 16)] = plsc.load_gather(
                        gbuf, [bidx[g], ev]
                    )

            stores[b] = pltpu.async_copy(
                trans, out_hbm.at[f, :, bb], ssems[b % 2]
            )
        stores[NBUF - 2].wait()
        stores[NBUF - 1].wait()
        return carry

    lax.fori_loop(0, TILES_PER_W // NBUF, lambda i, c: body(i * NBUF, c), 0)


def kernel(ids, table):
    idsf = ids.T.reshape(-1)
    out5 = _gather_kernel(table, idsf)
    return out5.transpose(2, 4, 0, 1, 3).reshape(BATCH, FIELDS, EMBED_DIM)


# speedup vs baseline: 1.0679x; 1.0014x over previous
import functools

import jax
import jax.numpy as jnp
from jax import lax
from jax.experimental import pallas as pl
from jax.experimental.pallas import tpu as pltpu
from jax.experimental.pallas import tpu_sc as plsc

BATCH = 16384
FIELDS = 26
EMBED_DIM = 32
VOCAB = 1000000

NUM_CORES = 2
NUM_SUBCORES = 16
NW = NUM_CORES * NUM_SUBCORES
BLK = 128
NBB = BATCH // BLK
BB_PER_W = NBB // NW
IDS_PER_W = BB_PER_W * BLK
TILES_PER_W = FIELDS * BB_PER_W
NBUF = 4

_mesh = plsc.VectorSubcoreMesh(
    core_axis_name="c",
    subcore_axis_name="s",
    num_cores=NUM_CORES,
    num_subcores=NUM_SUBCORES,
)


@functools.partial(
    pl.kernel,
    mesh=_mesh,
    out_type=jax.ShapeDtypeStruct((FIELDS, 4, NBB, 8, BLK), jnp.float32),
    scratch_types=[
        pltpu.VMEM((FIELDS * IDS_PER_W,), jnp.int32),
        [pltpu.VMEM((BLK, EMBED_DIM), jnp.float32) for _ in range(NBUF)],
        [pltpu.VMEM((4, 8, BLK), jnp.float32) for _ in range(2)],
        [pltpu.SemaphoreType.DMA for _ in range(NBUF)],
        [pltpu.SemaphoreType.DMA for _ in range(2)],
    ],
    compiler_params=pltpu.CompilerParams(
        use_tc_tiling_on_sc=False, needs_layout_passes=False
    ),
)
def _gather_kernel(t_hbm, idsf_hbm, out_hbm, idsv, gbufs, transb, gsems, ssems):
    wid = lax.axis_index("s") * NUM_CORES + lax.axis_index("c")
    for f in range(FIELDS):
        pltpu.sync_copy(
            idsf_hbm.at[pl.ds(f * BATCH + wid * IDS_PER_W, IDS_PER_W)],
            idsv.at[pl.ds(f * IDS_PER_W, IDS_PER_W)],
        )

    bidx = [lax.iota(jnp.int32, 16) + g * 16 for g in range(8)]
    eidx = [jnp.full((16,), e, jnp.int32) for e in range(EMBED_DIM)]

    def body(t0, carry):
        gathers = []
        for b in range(NBUF):
            t = t0 + b
            f = t // BB_PER_W
            bl = t % BB_PER_W
            gathers.append(
                pltpu.async_copy(
                    t_hbm.at[idsv.at[pl.ds(f * IDS_PER_W + bl * BLK, BLK)]],
                    gbufs[b],
                    gsems[b],
                )
            )
        stores = {}
        for b in range(NBUF):
            t = t0 + b
            f = t // BB_PER_W
            bb = wid * BB_PER_W + t % BB_PER_W
            gathers[b].wait()
            gbuf = gbufs[b]
            trans = transb[b % 2]
            if b >= 2:
                stores[b - 2].wait()

            @plsc.parallel_loop(0, EMBED_DIM, step=1, unroll=16)
            def _transpose(e):
                ev = jnp.full((16,), 1, jnp.int32) * e
                eb_i = lax.shift_right_logical(e, 3)
                es_i = jnp.bitwise_and(e, 7)
                for g in range(8):
                    trans[eb_i, es_i, pl.ds(g * 16, 16)] = plsc.load_gather(
                        gbuf, [bidx[g], ev]
                    )

            stores[b] = pltpu.async_copy(
                trans, out_hbm.at[f, :, bb], ssems[b % 2]
            )
        stores[NBUF - 2].wait()
        stores[NBUF - 1].wait()
        return carry

    lax.fori_loop(0, TILES_PER_W // NBUF, lambda i, c: body(i * NBUF, c), 0)


def kernel(ids, table):
    idsf = ids.T.reshape(-1)
    out5 = _gather_kernel(table, idsf)
    return out5.transpose(2, 4, 0, 1, 3).reshape(BATCH, FIELDS, EMBED_DIM)
